# baseline (speedup 1.0000x reference)
import dataclasses
import functools

import jax
import jax.numpy as jnp
from jax import lax
from jax.experimental import pallas as pl
from jax.experimental.pallas import tpu as pltpu
from jax.experimental.pallas import tpu_sc as plsc

NC = 2
NS = 16
L = 16
K = 80
KA = 128
BS = 1024

THETA = (0.5, 0.3, 0.1, 0.05, 0.05)

_SC_PARAMS = pltpu.CompilerParams()
if "needs_layout_passes" in pltpu.CompilerParams.__dataclass_fields__:
    _SC_PARAMS = dataclasses.replace(_SC_PARAMS, needs_layout_passes=False)


def _sc_degree(npad, ept, k):
    nr = npad // 128
    nchunks = ept // k
    mesh = plsc.VectorSubcoreMesh(core_axis_name="c", subcore_axis_name="s")

    @functools.partial(
        pl.kernel,
        mesh=mesh,
        compiler_params=_SC_PARAMS,
        out_type=jax.ShapeDtypeStruct((2 * nr, 128), jnp.float32),
        scratch_types=[
            pltpu.VMEM((1, k), jnp.int32),
            pltpu.VMEM((nr, 128), jnp.float32),
            pltpu.VMEM((1, nr), jnp.int32),
            pltpu.VMEM_SHARED((nr, 128), jnp.float32),
            pltpu.SemaphoreType.DMA,
        ],
    )
    def deg_kernel(row_hbm, out_hbm, idx_v, hist_v, iden_v, acc_sh, sem):
        cid = lax.axis_index("c")
        sid = lax.axis_index("s")

        @pl.loop(0, nr)
        def _(i):
            @pl.loop(0, 128, step=L)
            def _(j):
                hist_v[i, pl.ds(j, L)] = jnp.zeros((L,), jnp.float32)

        @pl.loop(0, nr, step=L)
        def _(i):
            iden_v[0, pl.ds(i, L)] = lax.iota(jnp.int32, L) + i

        @pl.when(sid < nr // 8)
        def _():
            pltpu.sync_copy(hist_v.at[pl.ds(sid * 8, 8)],
                            acc_sh.at[pl.ds(sid * 8, 8)])
        plsc.subcore_barrier()

        base = (cid * NS + sid) * ept
        ones16 = jnp.full((L,), 1.0, jnp.float32)

        @pl.loop(0, nchunks)
        def _(i):
            pltpu.sync_copy(row_hbm.at[pl.ds(base + i * k, k)], idx_v.at[0])

            @pl.loop(0, k, step=L)
            def _(j):
                idx = idx_v[0, pl.ds(j, L)]
                r = lax.shift_right_logical(idx, 7)
                c = lax.bitwise_and(idx, 127)
                plsc.addupdate_scatter(hist_v, [r, c], ones16)

        pltpu.sync_copy(hist_v, acc_sh.at[iden_v.at[0]], add=True)
        plsc.subcore_barrier()

        @pl.when(sid < nr // 8)
        def _():
            pltpu.sync_copy(acc_sh.at[pl.ds(sid * 8, 8)],
                            out_hbm.at[pl.ds(cid * nr + sid * 8, 8)])

    return deg_kernel


def _sc_apply(npad, d, cpt, k):
    rpt = npad // NS
    mesh = plsc.VectorSubcoreMesh(core_axis_name="c", subcore_axis_name="s")

    @functools.partial(
        pl.kernel,
        mesh=mesh,
        compiler_params=_SC_PARAMS,
        out_type=jax.ShapeDtypeStruct((2 * npad, d), jnp.float32),
        scratch_types=[
            pltpu.VMEM((1, k), jnp.int32),
            pltpu.VMEM((1, k), jnp.int32),
            pltpu.VMEM((1, k), jnp.int32),
            pltpu.VMEM((1, k), jnp.int32),
            pltpu.VMEM((k, d), jnp.float32),
            pltpu.VMEM((k, d), jnp.float32),
            pltpu.VMEM_SHARED((npad, d), jnp.float32),
            pltpu.SemaphoreType.DMA,
            pltpu.SemaphoreType.DMA,
            pltpu.SemaphoreType.DMA,
            pltpu.SemaphoreType.DMA,
            pltpu.SemaphoreType.DMA,
            pltpu.SemaphoreType.DMA,
        ],
    )
    def apply_kernel(s_hbm, col_hbm, row_hbm, out_hbm,
                     c0, r0, c1, r1, buf0, buf1, acc_sh,
                     gi0, gi1, g0, g1, s0, s1):
        cid = lax.axis_index("c")
        sid = lax.axis_index("s")
        base = (cid * NS + sid) * cpt * k

        def idx_load(cb, rb, sem, i):
            pltpu.async_copy(col_hbm.at[pl.ds(base + i * k, k)], cb.at[0], sem)
            pltpu.async_copy(row_hbm.at[pl.ds(base + i * k, k)], rb.at[0], sem)

        def idx_wait(cb, rb, sem):
            pltpu.make_async_copy(col_hbm.at[pl.ds(0, k)], cb.at[0], sem).wait()
            pltpu.make_async_copy(row_hbm.at[pl.ds(0, k)], rb.at[0], sem).wait()

        def gather_start(cb, buf, sem):
            pltpu.async_copy(s_hbm.at[cb.at[0]], buf, sem)

        def gather_wait(buf, sem):
            pltpu.make_async_copy(s_hbm.at[pl.ds(0, k)], buf, sem).wait()

        def scatter_start(buf, rb, sem):
            pltpu.async_copy(buf, acc_sh.at[rb.at[0]], sem, add=True)

        def scatter_wait(buf, sem):
            pltpu.make_async_copy(buf, acc_sh.at[pl.ds(0, k)], sem).wait()

        idx_load(c0, r0, gi0, 0)
        idx_load(c1, r1, gi1, 1)

        @pl.loop(0, k)
        def _(i):
            @pl.loop(0, d, step=L)
            def _(j):
                buf0[i, pl.ds(j, L)] = jnp.zeros((L,), jnp.float32)

        rz = sid * rpt

        @pl.loop(0, rpt, step=k)
        def _(r):
            pltpu.sync_copy(buf0, acc_sh.at[pl.ds(rz + r, k)])

        plsc.subcore_barrier()

        idx_wait(c0, r0, gi0)
        gather_start(c0, buf0, g0)

        @pl.loop(0, cpt, step=2)
        def _(i):
            gather_wait(buf0, g0)
            scatter_start(buf0, r0, s0)
            idx_wait(c1, r1, gi1)
            gather_start(c1, buf1, g1)
            scatter_wait(buf0, s0)

            @pl.when(i + 2 < cpt)
            def _():
                idx_load(c0, r0, gi0, i + 2)

            gather_wait(buf1, g1)
            scatter_start(buf1, r1, s1)

            @pl.when(i + 2 < cpt)
            def _():
                idx_wait(c0, r0, gi0)
                gather_start(c0, buf0, g0)

            scatter_wait(buf1, s1)

            @pl.when(i + 3 < cpt)
            def _():
                idx_load(c1, r1, gi1, i + 3)

        plsc.subcore_barrier()
        pltpu.sync_copy(acc_sh.at[pl.ds(rz, rpt)],
                        out_hbm.at[pl.ds(cid * npad + rz, rpt)])

    return apply_kernel


def _tc_init(npad, d, theta0):
    nb = npad // BS
    nr = npad // 128
    rpb = BS // 128

    def body(h_ref, d0_ref, d1_ref, dinv_ref, s_ref, oa_ref):
        deg = d0_ref[...] + d1_ref[...]
        dinv8 = jnp.where(deg > 0, lax.rsqrt(deg), 0.0)
        jrow = lax.broadcasted_iota(jnp.int32, (BS, rpb), 0) // 128
        sel = (jrow == lax.broadcasted_iota(jnp.int32, (BS, rpb), 1))
        spread = jax.lax.dot_general(
            sel.astype(jnp.float32), dinv8,
            dimension_numbers=(((1,), (0,)), ((), ())),
            preferred_element_type=jnp.float32)
        jcol = lax.broadcasted_iota(jnp.int32, (BS, 128), 0) % 128
        mask = (jcol == lax.broadcasted_iota(jnp.int32, (BS, 128), 1))
        dinv_col = jnp.sum(jnp.where(mask, spread, 0.0), axis=1,
                           keepdims=True)
        dinv_blk = lax.broadcast_in_dim(dinv_col, (BS, d), (0, 1))
        dinv_ref[...] = dinv_blk
        hb = h_ref[...]
        s_ref[...] = dinv_blk * hb
        oa_ref[...] = theta0 * hb

    return pl.pallas_call(
        body,
        grid=(nb,),
        in_specs=[
            pl.BlockSpec((BS, d), lambda i: (i, 0)),
            pl.BlockSpec((rpb, 128), lambda i: (i, 0)),
            pl.BlockSpec((rpb, 128), lambda i: (i + nb, 0)),
        ],
        out_specs=[
            pl.BlockSpec((BS, d), lambda i: (i, 0)),
            pl.BlockSpec((BS, d), lambda i: (i, 0)),
            pl.BlockSpec((BS, d), lambda i: (i, 0)),
        ],
        out_shape=[
            jax.ShapeDtypeStruct((npad, d), jnp.float32),
            jax.ShapeDtypeStruct((npad, d), jnp.float32),
            jax.ShapeDtypeStruct((npad, d), jnp.float32),
        ],
    )


def _tc_combine(npad, d, theta_k):
    nb = npad // BS

    def body(f_ref, p0_ref, p1_ref, dinv_ref, oa_ref,
             fn_ref, oan_ref, sn_ref):
        dinv = dinv_ref[...]
        fn = f_ref[...] - dinv * (p0_ref[...] + p1_ref[...])
        fn_ref[...] = fn
        oan_ref[...] = oa_ref[...] + theta_k * fn
        sn_ref[...] = dinv * fn

    return pl.pallas_call(
        body,
        grid=(nb,),
        in_specs=[
            pl.BlockSpec((BS, d), lambda i: (i, 0)),
            pl.BlockSpec((BS, d), lambda i: (i, 0)),
            pl.BlockSpec((BS, d), lambda i: (i + nb, 0)),
            pl.BlockSpec((BS, d), lambda i: (i, 0)),
            pl.BlockSpec((BS, d), lambda i: (i, 0)),
        ],
        out_specs=[
            pl.BlockSpec((BS, d), lambda i: (i, 0)),
            pl.BlockSpec((BS, d), lambda i: (i, 0)),
            pl.BlockSpec((BS, d), lambda i: (i, 0)),
        ],
        out_shape=[
            jax.ShapeDtypeStruct((npad, d), jnp.float32),
            jax.ShapeDtypeStruct((npad, d), jnp.float32),
            jax.ShapeDtypeStruct((npad, d), jnp.float32),
        ],
    )


def kernel(h, edge_index):
    n, d = h.shape
    e = edge_index.shape[1]

    npad = ((n + NS * KA - 1) // (NS * KA)) * (NS * KA)
    echunk = NC * NS * KA * 2
    epad = ((e + echunk - 1) // echunk) * echunk
    cpt = epad // (NC * NS * KA)
    ept = epad // (NC * NS)

    row = edge_index[0]
    col = edge_index[1]
    if epad != e:
        row = jnp.concatenate(
            [row, jnp.full((epad - e,), npad - 1, jnp.int32)])
        col = jnp.concatenate([col, jnp.zeros((epad - e,), jnp.int32)])
    h_pad = jnp.pad(h, ((0, npad - n), (0, 0))) if npad != n else h

    deg_p = _sc_degree(npad, ept, K)(row)
    dinv, s, out = _tc_init(npad, d, THETA[0])(h_pad, deg_p, deg_p)

    sc_apply = _sc_apply(npad, d, cpt, KA)
    feat = h_pad
    for kk in range(1, len(THETA)):
        part = sc_apply(s, col, row)
        feat, out, s = _tc_combine(npad, d, THETA[kk])(
            feat, part, part, dinv, out)

    return out[:n]


# speedup vs baseline: 1.6986x; 1.6986x over previous
import dataclasses
import functools

import jax
import jax.numpy as jnp
from jax import lax
from jax.experimental import pallas as pl
from jax.experimental.pallas import tpu as pltpu
from jax.experimental.pallas import tpu_sc as plsc

NC = 2
NS = 16
L = 16
K = 80
KA = 64
BS = 1024

THETA = (0.5, 0.3, 0.1, 0.05, 0.05)

_SC_PARAMS = pltpu.CompilerParams()
if "needs_layout_passes" in pltpu.CompilerParams.__dataclass_fields__:
    _SC_PARAMS = dataclasses.replace(_SC_PARAMS, needs_layout_passes=False)


def _sc_degree(npad, ept, k):
    nr = npad // 128
    nchunks = ept // k
    mesh = plsc.VectorSubcoreMesh(core_axis_name="c", subcore_axis_name="s")

    @functools.partial(
        pl.kernel,
        mesh=mesh,
        compiler_params=_SC_PARAMS,
        out_type=jax.ShapeDtypeStruct((2 * nr, 128), jnp.float32),
        scratch_types=[
            pltpu.VMEM((1, k), jnp.int32),
            pltpu.VMEM((nr, 128), jnp.float32),
            pltpu.VMEM((1, nr), jnp.int32),
            pltpu.VMEM_SHARED((nr, 128), jnp.float32),
            pltpu.SemaphoreType.DMA,
        ],
    )
    def deg_kernel(row_hbm, out_hbm, idx_v, hist_v, iden_v, acc_sh, sem):
        cid = lax.axis_index("c")
        sid = lax.axis_index("s")

        @pl.loop(0, nr)
        def _(i):
            @pl.loop(0, 128, step=L)
            def _(j):
                hist_v[i, pl.ds(j, L)] = jnp.zeros((L,), jnp.float32)

        @pl.loop(0, nr, step=L)
        def _(i):
            iden_v[0, pl.ds(i, L)] = lax.iota(jnp.int32, L) + i

        @pl.when(sid < nr // 8)
        def _():
            pltpu.sync_copy(hist_v.at[pl.ds(sid * 8, 8)],
                            acc_sh.at[pl.ds(sid * 8, 8)])
        plsc.subcore_barrier()

        base = (cid * NS + sid) * ept
        ones16 = jnp.full((L,), 1.0, jnp.float32)

        @pl.loop(0, nchunks)
        def _(i):
            pltpu.sync_copy(row_hbm.at[pl.ds(base + i * k, k)], idx_v.at[0])

            @pl.loop(0, k, step=L)
            def _(j):
                idx = idx_v[0, pl.ds(j, L)]
                r = lax.shift_right_logical(idx, 7)
                c = lax.bitwise_and(idx, 127)
                plsc.addupdate_scatter(hist_v, [r, c], ones16)

        pltpu.sync_copy(hist_v, acc_sh.at[iden_v.at[0]], add=True)
        plsc.subcore_barrier()

        @pl.when(sid < nr // 8)
        def _():
            pltpu.sync_copy(acc_sh.at[pl.ds(sid * 8, 8)],
                            out_hbm.at[pl.ds(cid * nr + sid * 8, 8)])

    return deg_kernel


def _sc_apply(npad, d, cpt, k):
    rpt = npad // NS
    mesh = plsc.VectorSubcoreMesh(core_axis_name="c", subcore_axis_name="s")

    @functools.partial(
        pl.kernel,
        mesh=mesh,
        compiler_params=_SC_PARAMS,
        out_type=jax.ShapeDtypeStruct((2 * npad, d), jnp.float32),
        scratch_types=[
            pltpu.VMEM((1, k), jnp.int32),
            pltpu.VMEM((1, k), jnp.int32),
            pltpu.VMEM((1, k), jnp.int32),
            pltpu.VMEM((1, k), jnp.int32),
            pltpu.VMEM((k, d), jnp.float32),
            pltpu.VMEM((k, d), jnp.float32),
            pltpu.VMEM_SHARED((npad, d), jnp.float32),
            pltpu.SemaphoreType.DMA,
            pltpu.SemaphoreType.DMA,
            pltpu.SemaphoreType.DMA,
            pltpu.SemaphoreType.DMA,
            pltpu.SemaphoreType.DMA,
            pltpu.SemaphoreType.DMA,
        ],
    )
    def apply_kernel(s_hbm, col_hbm, row_hbm, out_hbm,
                     c0, r0, c1, r1, buf0, buf1, acc_sh,
                     gi0, gi1, g0, g1, s0, s1):
        cid = lax.axis_index("c")
        sid = lax.axis_index("s")
        base = (cid * NS + sid) * cpt * k

        def idx_load(cb, rb, sem, i):
            pltpu.async_copy(col_hbm.at[pl.ds(base + i * k, k)], cb.at[0], sem)
            pltpu.async_copy(row_hbm.at[pl.ds(base + i * k, k)], rb.at[0], sem)

        def idx_wait(cb, rb, sem):
            pltpu.make_async_copy(col_hbm.at[pl.ds(0, k)], cb.at[0], sem).wait()
            pltpu.make_async_copy(row_hbm.at[pl.ds(0, k)], rb.at[0], sem).wait()

        def gather_start(cb, buf, sem):
            pltpu.async_copy(s_hbm.at[cb.at[0]], buf, sem)

        def gather_wait(buf, sem):
            pltpu.make_async_copy(s_hbm.at[pl.ds(0, k)], buf, sem).wait()

        def scatter_start(buf, rb, sem):
            pltpu.async_copy(buf, acc_sh.at[rb.at[0]], sem, add=True)

        def scatter_wait(buf, sem):
            pltpu.make_async_copy(buf, acc_sh.at[pl.ds(0, k)], sem).wait()

        idx_load(c0, r0, gi0, 0)
        idx_load(c1, r1, gi1, 1)

        @pl.loop(0, k)
        def _(i):
            @pl.loop(0, d, step=L)
            def _(j):
                buf0[i, pl.ds(j, L)] = jnp.zeros((L,), jnp.float32)

        rz = sid * rpt

        @pl.loop(0, rpt, step=k)
        def _(r):
            pltpu.sync_copy(buf0, acc_sh.at[pl.ds(rz + r, k)])

        plsc.subcore_barrier()

        idx_wait(c0, r0, gi0)
        gather_start(c0, buf0, g0)

        @pl.loop(0, cpt, step=2)
        def _(i):
            gather_wait(buf0, g0)
            scatter_start(buf0, r0, s0)
            idx_wait(c1, r1, gi1)
            gather_start(c1, buf1, g1)
            scatter_wait(buf0, s0)

            @pl.when(i + 2 < cpt)
            def _():
                idx_load(c0, r0, gi0, i + 2)

            gather_wait(buf1, g1)
            scatter_start(buf1, r1, s1)

            @pl.when(i + 2 < cpt)
            def _():
                idx_wait(c0, r0, gi0)
                gather_start(c0, buf0, g0)

            scatter_wait(buf1, s1)

            @pl.when(i + 3 < cpt)
            def _():
                idx_load(c1, r1, gi1, i + 3)

        plsc.subcore_barrier()
        pltpu.sync_copy(acc_sh.at[pl.ds(rz, rpt)],
                        out_hbm.at[pl.ds(cid * npad + rz, rpt)])

    return apply_kernel


def _tc_init(npad, d, theta0):
    nb = npad // BS
    nr = npad // 128
    rpb = BS // 128

    def body(h_ref, d0_ref, d1_ref, dinv_ref, s_ref, oa_ref):
        deg = d0_ref[...] + d1_ref[...]
        dinv8 = jnp.where(deg > 0, lax.rsqrt(deg), 0.0)
        jrow = lax.broadcasted_iota(jnp.int32, (BS, rpb), 0) // 128
        sel = (jrow == lax.broadcasted_iota(jnp.int32, (BS, rpb), 1))
        spread = jax.lax.dot_general(
            sel.astype(jnp.float32), dinv8,
            dimension_numbers=(((1,), (0,)), ((), ())),
            preferred_element_type=jnp.float32)
        jcol = lax.broadcasted_iota(jnp.int32, (BS, 128), 0) % 128
        mask = (jcol == lax.broadcasted_iota(jnp.int32, (BS, 128), 1))
        dinv_col = jnp.sum(jnp.where(mask, spread, 0.0), axis=1,
                           keepdims=True)
        dinv_blk = lax.broadcast_in_dim(dinv_col, (BS, d), (0, 1))
        dinv_ref[...] = dinv_blk
        hb = h_ref[...]
        s_ref[...] = dinv_blk * hb
        oa_ref[...] = theta0 * hb

    return pl.pallas_call(
        body,
        grid=(nb,),
        in_specs=[
            pl.BlockSpec((BS, d), lambda i: (i, 0)),
            pl.BlockSpec((rpb, 128), lambda i: (i, 0)),
            pl.BlockSpec((rpb, 128), lambda i: (i + nb, 0)),
        ],
        out_specs=[
            pl.BlockSpec((BS, d), lambda i: (i, 0)),
            pl.BlockSpec((BS, d), lambda i: (i, 0)),
            pl.BlockSpec((BS, d), lambda i: (i, 0)),
        ],
        out_shape=[
            jax.ShapeDtypeStruct((npad, d), jnp.float32),
            jax.ShapeDtypeStruct((npad, d), jnp.float32),
            jax.ShapeDtypeStruct((npad, d), jnp.float32),
        ],
    )


def _tc_combine(npad, d, theta_k):
    nb = npad // BS

    def body(f_ref, p0_ref, p1_ref, dinv_ref, oa_ref,
             fn_ref, oan_ref, sn_ref):
        dinv = dinv_ref[...]
        fn = f_ref[...] - dinv * (p0_ref[...] + p1_ref[...])
        fn_ref[...] = fn
        oan_ref[...] = oa_ref[...] + theta_k * fn
        sn_ref[...] = dinv * fn

    return pl.pallas_call(
        body,
        grid=(nb,),
        in_specs=[
            pl.BlockSpec((BS, d), lambda i: (i, 0)),
            pl.BlockSpec((BS, d), lambda i: (i, 0)),
            pl.BlockSpec((BS, d), lambda i: (i + nb, 0)),
            pl.BlockSpec((BS, d), lambda i: (i, 0)),
            pl.BlockSpec((BS, d), lambda i: (i, 0)),
        ],
        out_specs=[
            pl.BlockSpec((BS, d), lambda i: (i, 0)),
            pl.BlockSpec((BS, d), lambda i: (i, 0)),
            pl.BlockSpec((BS, d), lambda i: (i, 0)),
        ],
        out_shape=[
            jax.ShapeDtypeStruct((npad, d), jnp.float32),
            jax.ShapeDtypeStruct((npad, d), jnp.float32),
            jax.ShapeDtypeStruct((npad, d), jnp.float32),
        ],
    )


def kernel(h, edge_index):
    n, d = h.shape
    e = edge_index.shape[1]

    npad = ((n + NS * KA - 1) // (NS * KA)) * (NS * KA)
    echunk = NC * NS * KA * 2
    epad = ((e + echunk - 1) // echunk) * echunk
    cpt = epad // (NC * NS * KA)
    ept = epad // (NC * NS)

    row = edge_index[0]
    col = edge_index[1]
    if epad != e:
        row = jnp.concatenate(
            [row, jnp.full((epad - e,), npad - 1, jnp.int32)])
        col = jnp.concatenate([col, jnp.zeros((epad - e,), jnp.int32)])
    h_pad = jnp.pad(h, ((0, npad - n), (0, 0))) if npad != n else h

    deg_p = _sc_degree(npad, ept, K)(row)
    dinv, s, out = _tc_init(npad, d, THETA[0])(h_pad, deg_p, deg_p)

    sc_apply = _sc_apply(npad, d, cpt, KA)
    feat = h_pad
    for kk in range(1, len(THETA)):
        part = sc_apply(s, col, row)
        feat, out, s = _tc_combine(npad, d, THETA[kk])(
            feat, part, part, dinv, out)

    return out[:n]


# speedup vs baseline: 2.0407x; 1.2014x over previous
import dataclasses
import functools

import jax
import jax.numpy as jnp
from jax import lax
from jax.experimental import pallas as pl
from jax.experimental.pallas import tpu as pltpu
from jax.experimental.pallas import tpu_sc as plsc

NC = 2
NS = 16
L = 16
K = 80
KA = 80
BS = 1024

THETA = (0.5, 0.3, 0.1, 0.05, 0.05)

_SC_PARAMS = pltpu.CompilerParams()
if "needs_layout_passes" in pltpu.CompilerParams.__dataclass_fields__:
    _SC_PARAMS = dataclasses.replace(_SC_PARAMS, needs_layout_passes=False)


def _sc_degree(npad, ept, k):
    nr = npad // 128
    nchunks = ept // k
    mesh = plsc.VectorSubcoreMesh(core_axis_name="c", subcore_axis_name="s")

    @functools.partial(
        pl.kernel,
        mesh=mesh,
        compiler_params=_SC_PARAMS,
        out_type=jax.ShapeDtypeStruct((2 * nr, 128), jnp.float32),
        scratch_types=[
            pltpu.VMEM((1, k), jnp.int32),
            pltpu.VMEM((nr, 128), jnp.float32),
            pltpu.VMEM((1, nr), jnp.int32),
            pltpu.VMEM_SHARED((nr, 128), jnp.float32),
            pltpu.SemaphoreType.DMA,
        ],
    )
    def deg_kernel(row_hbm, out_hbm, idx_v, hist_v, iden_v, acc_sh, sem):
        cid = lax.axis_index("c")
        sid = lax.axis_index("s")

        @pl.loop(0, nr)
        def _(i):
            @pl.loop(0, 128, step=L)
            def _(j):
                hist_v[i, pl.ds(j, L)] = jnp.zeros((L,), jnp.float32)

        @pl.loop(0, nr, step=L)
        def _(i):
            iden_v[0, pl.ds(i, L)] = lax.iota(jnp.int32, L) + i

        @pl.when(sid < nr // 8)
        def _():
            pltpu.sync_copy(hist_v.at[pl.ds(sid * 8, 8)],
                            acc_sh.at[pl.ds(sid * 8, 8)])
        plsc.subcore_barrier()

        base = (cid * NS + sid) * ept
        ones16 = jnp.full((L,), 1.0, jnp.float32)

        @pl.loop(0, nchunks)
        def _(i):
            pltpu.sync_copy(row_hbm.at[pl.ds(base + i * k, k)], idx_v.at[0])

            @pl.loop(0, k, step=L)
            def _(j):
                idx = idx_v[0, pl.ds(j, L)]
                r = lax.shift_right_logical(idx, 7)
                c = lax.bitwise_and(idx, 127)
                plsc.addupdate_scatter(hist_v, [r, c], ones16)

        pltpu.sync_copy(hist_v, acc_sh.at[iden_v.at[0]], add=True)
        plsc.subcore_barrier()

        @pl.when(sid < nr // 8)
        def _():
            pltpu.sync_copy(acc_sh.at[pl.ds(sid * 8, 8)],
                            out_hbm.at[pl.ds(cid * nr + sid * 8, 8)])

    return deg_kernel


def _sc_apply(npad, d, cpt, k):
    rpt = npad // NS
    mesh = plsc.VectorSubcoreMesh(core_axis_name="c", subcore_axis_name="s")

    @functools.partial(
        pl.kernel,
        mesh=mesh,
        compiler_params=_SC_PARAMS,
        out_type=jax.ShapeDtypeStruct((2 * npad, d), jnp.float32),
        scratch_types=[
            pltpu.VMEM((1, k), jnp.int32),
            pltpu.VMEM((1, k), jnp.int32),
            pltpu.VMEM((1, k), jnp.int32),
            pltpu.VMEM((1, k), jnp.int32),
            pltpu.VMEM((k, d), jnp.float32),
            pltpu.VMEM((k, d), jnp.float32),
            pltpu.VMEM_SHARED((npad, d), jnp.float32),
            pltpu.SemaphoreType.DMA,
            pltpu.SemaphoreType.DMA,
            pltpu.SemaphoreType.DMA,
            pltpu.SemaphoreType.DMA,
            pltpu.SemaphoreType.DMA,
            pltpu.SemaphoreType.DMA,
        ],
    )
    def apply_kernel(s_hbm, col_hbm, row_hbm, out_hbm,
                     c0, r0, c1, r1, buf0, buf1, acc_sh,
                     gi0, gi1, g0, g1, s0, s1):
        cid = lax.axis_index("c")
        sid = lax.axis_index("s")
        base = (cid * NS + sid) * cpt * k

        def idx_load(cb, rb, sem, i):
            pltpu.async_copy(col_hbm.at[pl.ds(base + i * k, k)], cb.at[0], sem)
            pltpu.async_copy(row_hbm.at[pl.ds(base + i * k, k)], rb.at[0], sem)

        def idx_wait(cb, rb, sem):
            pltpu.make_async_copy(col_hbm.at[pl.ds(0, k)], cb.at[0], sem).wait()
            pltpu.make_async_copy(row_hbm.at[pl.ds(0, k)], rb.at[0], sem).wait()

        def gather_start(cb, buf, sem):
            pltpu.async_copy(s_hbm.at[cb.at[0]], buf, sem)

        def gather_wait(buf, sem):
            pltpu.make_async_copy(s_hbm.at[pl.ds(0, k)], buf, sem).wait()

        def scatter_start(buf, rb, sem):
            pltpu.async_copy(buf, acc_sh.at[rb.at[0]], sem, add=True)

        def scatter_wait(buf, sem):
            pltpu.make_async_copy(buf, acc_sh.at[pl.ds(0, k)], sem).wait()

        idx_load(c0, r0, gi0, 0)
        idx_load(c1, r1, gi1, 1)

        @pl.loop(0, k)
        def _(i):
            @pl.loop(0, d, step=L)
            def _(j):
                buf0[i, pl.ds(j, L)] = jnp.zeros((L,), jnp.float32)

        rz = sid * rpt

        @pl.loop(0, rpt, step=k)
        def _(r):
            pltpu.sync_copy(buf0, acc_sh.at[pl.ds(rz + r, k)])

        plsc.subcore_barrier()

        idx_wait(c0, r0, gi0)
        gather_start(c0, buf0, g0)

        @pl.loop(0, cpt, step=2)
        def _(i):
            gather_wait(buf0, g0)
            scatter_start(buf0, r0, s0)
            idx_wait(c1, r1, gi1)
            gather_start(c1, buf1, g1)
            scatter_wait(buf0, s0)

            @pl.when(i + 2 < cpt)
            def _():
                idx_load(c0, r0, gi0, i + 2)

            gather_wait(buf1, g1)
            scatter_start(buf1, r1, s1)

            @pl.when(i + 2 < cpt)
            def _():
                idx_wait(c0, r0, gi0)
                gather_start(c0, buf0, g0)

            scatter_wait(buf1, s1)

            @pl.when(i + 3 < cpt)
            def _():
                idx_load(c1, r1, gi1, i + 3)

        plsc.subcore_barrier()
        pltpu.sync_copy(acc_sh.at[pl.ds(rz, rpt)],
                        out_hbm.at[pl.ds(cid * npad + rz, rpt)])

    return apply_kernel


def _tc_init(npad, d, theta0):
    nb = npad // BS
    nr = npad // 128
    rpb = BS // 128

    def body(h_ref, d0_ref, d1_ref, dinv_ref, s_ref, oa_ref):
        deg = d0_ref[...] + d1_ref[...]
        dinv8 = jnp.where(deg > 0, lax.rsqrt(deg), 0.0)
        jrow = lax.broadcasted_iota(jnp.int32, (BS, rpb), 0) // 128
        sel = (jrow == lax.broadcasted_iota(jnp.int32, (BS, rpb), 1))
        spread = jax.lax.dot_general(
            sel.astype(jnp.float32), dinv8,
            dimension_numbers=(((1,), (0,)), ((), ())),
            preferred_element_type=jnp.float32)
        jcol = lax.broadcasted_iota(jnp.int32, (BS, 128), 0) % 128
        mask = (jcol == lax.broadcasted_iota(jnp.int32, (BS, 128), 1))
        dinv_col = jnp.sum(jnp.where(mask, spread, 0.0), axis=1,
                           keepdims=True)
        dinv_blk = lax.broadcast_in_dim(dinv_col, (BS, d), (0, 1))
        dinv_ref[...] = dinv_blk
        hb = h_ref[...]
        s_ref[...] = dinv_blk * hb
        oa_ref[...] = theta0 * hb

    return pl.pallas_call(
        body,
        grid=(nb,),
        in_specs=[
            pl.BlockSpec((BS, d), lambda i: (i, 0)),
            pl.BlockSpec((rpb, 128), lambda i: (i, 0)),
            pl.BlockSpec((rpb, 128), lambda i: (i + nb, 0)),
        ],
        out_specs=[
            pl.BlockSpec((BS, d), lambda i: (i, 0)),
            pl.BlockSpec((BS, d), lambda i: (i, 0)),
            pl.BlockSpec((BS, d), lambda i: (i, 0)),
        ],
        out_shape=[
            jax.ShapeDtypeStruct((npad, d), jnp.float32),
            jax.ShapeDtypeStruct((npad, d), jnp.float32),
            jax.ShapeDtypeStruct((npad, d), jnp.float32),
        ],
    )


def _tc_combine(npad, d, theta_k):
    nb = npad // BS

    def body(f_ref, p0_ref, p1_ref, dinv_ref, oa_ref,
             fn_ref, oan_ref, sn_ref):
        dinv = dinv_ref[...]
        fn = f_ref[...] - dinv * (p0_ref[...] + p1_ref[...])
        fn_ref[...] = fn
        oan_ref[...] = oa_ref[...] + theta_k * fn
        sn_ref[...] = dinv * fn

    return pl.pallas_call(
        body,
        grid=(nb,),
        in_specs=[
            pl.BlockSpec((BS, d), lambda i: (i, 0)),
            pl.BlockSpec((BS, d), lambda i: (i, 0)),
            pl.BlockSpec((BS, d), lambda i: (i + nb, 0)),
            pl.BlockSpec((BS, d), lambda i: (i, 0)),
            pl.BlockSpec((BS, d), lambda i: (i, 0)),
        ],
        out_specs=[
            pl.BlockSpec((BS, d), lambda i: (i, 0)),
            pl.BlockSpec((BS, d), lambda i: (i, 0)),
            pl.BlockSpec((BS, d), lambda i: (i, 0)),
        ],
        out_shape=[
            jax.ShapeDtypeStruct((npad, d), jnp.float32),
            jax.ShapeDtypeStruct((npad, d), jnp.float32),
            jax.ShapeDtypeStruct((npad, d), jnp.float32),
        ],
    )


def kernel(h, edge_index):
    n, d = h.shape
    e = edge_index.shape[1]

    npad = ((n + NS * KA - 1) // (NS * KA)) * (NS * KA)
    echunk = NC * NS * KA * 2
    epad = ((e + echunk - 1) // echunk) * echunk
    cpt = epad // (NC * NS * KA)
    ept = epad // (NC * NS)

    row = edge_index[0]
    col = edge_index[1]
    if epad != e:
        row = jnp.concatenate(
            [row, jnp.full((epad - e,), npad - 1, jnp.int32)])
        col = jnp.concatenate([col, jnp.zeros((epad - e,), jnp.int32)])
    h_pad = jnp.pad(h, ((0, npad - n), (0, 0))) if npad != n else h

    deg_p = _sc_degree(npad, ept, K)(row)
    dinv, s, out = _tc_init(npad, d, THETA[0])(h_pad, deg_p, deg_p)

    sc_apply = _sc_apply(npad, d, cpt, KA)
    feat = h_pad
    for kk in range(1, len(THETA)):
        part = sc_apply(s, col, row)
        feat, out, s = _tc_combine(npad, d, THETA[kk])(
            feat, part, part, dinv, out)

    return out[:n]
